# baseline (speedup 1.0000x reference)
import functools

import jax
import jax.numpy as jnp
from jax import lax
from jax.experimental import pallas as pl
from jax.experimental.pallas import tpu as pltpu
from jax.experimental.pallas import tpu_sc as plsc

_EMBED = 64
_BATCH = 16384
_ROWS = 1000000
_LANE = 128
_DEPTH = 2


def _build():
    info = plsc.get_sparse_core_info()
    nc, ns, nl = info.num_cores, info.num_subcores, info.num_lanes
    nw = nc * ns
    b_per_w = _BATCH // nw
    n_blocks = b_per_w // _LANE
    mesh = plsc.VectorSubcoreMesh(core_axis_name="c", subcore_axis_name="s")

    @functools.partial(
        pl.kernel,
        mesh=mesh,
        compiler_params=pltpu.CompilerParams(needs_layout_passes=False),
        out_type=jax.ShapeDtypeStruct((_EMBED, _BATCH), jnp.float32),
        scratch_types=[
            pltpu.VMEM((b_per_w + 32,), jnp.int32),
            pltpu.VMEM((b_per_w + 32,), jnp.int32),
            pltpu.VMEM((_DEPTH, _EMBED, _LANE), jnp.float32),
            pltpu.VMEM((_DEPTH, _EMBED, _LANE), jnp.float32),
            pltpu.VMEM((_EMBED, _LANE), jnp.float32),
            pltpu.VMEM((_EMBED, _LANE), jnp.float32),
            [pltpu.SemaphoreType.DMA] * _DEPTH,
            pltpu.SemaphoreType.DMA,
            pltpu.SemaphoreType.DMA,
        ],
    )
    def gmf(uidx_hbm, iidx_hbm, utab_hbm, itab_hbm, out_hbm,
            uidx_v, iidx_v, uw, iw, prod0, prod1, wsem, osem0, osem1):
        wid = lax.axis_index("s") * nc + lax.axis_index("c")
        base = wid * b_per_w
        pltpu.sync_copy(uidx_hbm.at[pl.ds(base, b_per_w)],
                        uidx_v.at[pl.ds(0, b_per_w)])
        pltpu.sync_copy(iidx_hbm.at[pl.ds(base, b_per_w)],
                        iidx_v.at[pl.ds(0, b_per_w)])

        prod = (prod0, prod1)
        osem = (osem0, osem1)
        iotas = tuple(lax.iota(jnp.int32, nl) + j * nl
                      for j in range(_EMBED // nl))

        def issue(u_idx, i_idx, slot):
            uo = pl.multiple_of(lax.shift_right_logical(u_idx, 7) * _LANE, 128)
            io = pl.multiple_of(lax.shift_right_logical(i_idx, 7) * _LANE, 128)
            pltpu.async_copy(utab_hbm.at[:, pl.ds(uo, _LANE)], uw.at[slot],
                             wsem[slot])
            pltpu.async_copy(itab_hbm.at[:, pl.ds(io, _LANE)], iw.at[slot],
                             wsem[slot])

        v0u = uidx_v[pl.ds(0, nl)]
        v0i = iidx_v[pl.ds(0, nl)]
        for s in range(_DEPTH):
            issue(v0u[s], v0i[s], s)

        def make_body(blk):
            p = blk & 1

            def body(k2, carry):
                k = blk * (_LANE // nl) + k2
                go = k * nl
                uv = uidx_v[pl.ds(go, nl)]
                un = uidx_v[pl.ds(go + nl, nl)]
                iv = iidx_v[pl.ds(go, nl)]
                inx = iidx_v[pl.ds(go + nl, nl)]
                for i in range(nl):
                    ci = go + i
                    s = i % _DEPTH
                    pltpu.make_async_copy(utab_hbm.at[:, pl.ds(0, _LANE)],
                                          uw.at[s], wsem[s]).wait()
                    pltpu.make_async_copy(itab_hbm.at[:, pl.ds(0, _LANE)],
                                          iw.at[s], wsem[s]).wait()
                    lu = jnp.full((nl,), lax.bitwise_and(uv[i], 127),
                                  jnp.int32)
                    li = jnp.full((nl,), lax.bitwise_and(iv[i], 127),
                                  jnp.int32)
                    lo = jnp.full((nl,), lax.bitwise_and(ci, 127), jnp.int32)
                    for j in range(_EMBED // nl):
                        gu = plsc.load_gather(uw.at[s], [iotas[j], lu])
                        gi = plsc.load_gather(iw.at[s], [iotas[j], li])
                        plsc.store_scatter(prod[p], [iotas[j], lo], gu * gi)
                    nu = un[(i + _DEPTH) % nl] if i + _DEPTH >= nl else uv[i + _DEPTH]
                    ni = inx[(i + _DEPTH) % nl] if i + _DEPTH >= nl else iv[i + _DEPTH]

                    @pl.when(ci + _DEPTH < b_per_w)
                    def _():
                        issue(nu, ni, s)
                return carry
            return body

        for blk in range(n_blocks):
            p = blk & 1
            if blk >= 2:
                pltpu.make_async_copy(out_hbm.at[:, pl.ds(0, _LANE)], prod[p],
                                      osem[p]).wait()
            lax.fori_loop(0, _LANE // nl, make_body(blk), 0)
            pltpu.async_copy(prod[p],
                             out_hbm.at[:, pl.ds(base + blk * _LANE, _LANE)],
                             osem[p])

        pltpu.make_async_copy(out_hbm.at[:, pl.ds(0, _LANE)], prod0,
                              osem0).wait()
        pltpu.make_async_copy(out_hbm.at[:, pl.ds(0, _LANE)], prod1,
                              osem1).wait()

    return gmf


_gmf = _build()


def kernel(user_indices, item_indices, user_table, item_table):
    uidx = user_indices.astype(jnp.int32)
    iidx = item_indices.astype(jnp.int32)
    outT = _gmf(uidx, iidx, user_table.T, item_table.T)
    return outT.T


# speedup vs baseline: 1.2444x; 1.2444x over previous
import functools

import jax
import jax.numpy as jnp
from jax import lax
from jax.experimental import pallas as pl
from jax.experimental.pallas import tpu as pltpu
from jax.experimental.pallas import tpu_sc as plsc

_EMBED = 64
_BATCH = 16384
_ROWS = 1000000
_LANE = 128
_DEPTH = 4


def _build():
    info = plsc.get_sparse_core_info()
    nc, ns, nl = info.num_cores, info.num_subcores, info.num_lanes
    nw = nc * ns
    b_per_w = _BATCH // nw
    n_blocks = b_per_w // _LANE
    mesh = plsc.VectorSubcoreMesh(core_axis_name="c", subcore_axis_name="s")

    @functools.partial(
        pl.kernel,
        mesh=mesh,
        compiler_params=pltpu.CompilerParams(needs_layout_passes=False),
        out_type=jax.ShapeDtypeStruct((_EMBED, _BATCH), jnp.float32),
        scratch_types=[
            pltpu.VMEM((b_per_w + 32,), jnp.int32),
            pltpu.VMEM((b_per_w + 32,), jnp.int32),
            pltpu.VMEM((_DEPTH, _EMBED, _LANE), jnp.float32),
            pltpu.VMEM((_DEPTH, _EMBED, _LANE), jnp.float32),
            pltpu.VMEM((_EMBED, _LANE), jnp.float32),
            pltpu.VMEM((_EMBED, _LANE), jnp.float32),
            [pltpu.SemaphoreType.DMA] * _DEPTH,
            pltpu.SemaphoreType.DMA,
            pltpu.SemaphoreType.DMA,
        ],
    )
    def gmf(uidx_hbm, iidx_hbm, utab_hbm, itab_hbm, out_hbm,
            uidx_v, iidx_v, uw, iw, prod0, prod1, wsem, osem0, osem1):
        wid = lax.axis_index("s") * nc + lax.axis_index("c")
        base = wid * b_per_w
        pltpu.sync_copy(uidx_hbm.at[pl.ds(base, b_per_w)],
                        uidx_v.at[pl.ds(0, b_per_w)])
        pltpu.sync_copy(iidx_hbm.at[pl.ds(base, b_per_w)],
                        iidx_v.at[pl.ds(0, b_per_w)])

        prod = (prod0, prod1)
        osem = (osem0, osem1)
        iotas = tuple(lax.iota(jnp.int32, nl) + j * nl
                      for j in range(_EMBED // nl))

        def issue(u_idx, i_idx, slot):
            uo = pl.multiple_of(lax.shift_right_logical(u_idx, 7) * _LANE, 128)
            io = pl.multiple_of(lax.shift_right_logical(i_idx, 7) * _LANE, 128)
            pltpu.async_copy(utab_hbm.at[:, pl.ds(uo, _LANE)], uw.at[slot],
                             wsem[slot])
            pltpu.async_copy(itab_hbm.at[:, pl.ds(io, _LANE)], iw.at[slot],
                             wsem[slot])

        v0u = uidx_v[pl.ds(0, nl)]
        v0i = iidx_v[pl.ds(0, nl)]
        for s in range(_DEPTH):
            issue(v0u[s], v0i[s], s)

        def make_body(blk):
            p = blk & 1

            def body(k2, carry):
                k = blk * (_LANE // nl) + k2
                go = k * nl
                uv = uidx_v[pl.ds(go, nl)]
                un = uidx_v[pl.ds(go + nl, nl)]
                iv = iidx_v[pl.ds(go, nl)]
                inx = iidx_v[pl.ds(go + nl, nl)]
                for i in range(nl):
                    ci = go + i
                    s = i % _DEPTH
                    pltpu.make_async_copy(utab_hbm.at[:, pl.ds(0, _LANE)],
                                          uw.at[s], wsem[s]).wait()
                    pltpu.make_async_copy(itab_hbm.at[:, pl.ds(0, _LANE)],
                                          iw.at[s], wsem[s]).wait()
                    lu = jnp.full((nl,), lax.bitwise_and(uv[i], 127),
                                  jnp.int32)
                    li = jnp.full((nl,), lax.bitwise_and(iv[i], 127),
                                  jnp.int32)
                    lo = jnp.full((nl,), lax.bitwise_and(ci, 127), jnp.int32)
                    for j in range(_EMBED // nl):
                        if j == 0:
                            gu = plsc.load_gather(uw.at[s], [iotas[j], lu])
                            gi = plsc.load_gather(iw.at[s], [iotas[j], li])
                            plsc.store_scatter(prod[p], [iotas[j], lo], gu * gi)
                    nu = un[(i + _DEPTH) % nl] if i + _DEPTH >= nl else uv[i + _DEPTH]
                    ni = inx[(i + _DEPTH) % nl] if i + _DEPTH >= nl else iv[i + _DEPTH]

                    @pl.when(ci + _DEPTH < b_per_w)
                    def _():
                        issue(nu, ni, s)
                return carry
            return body

        for blk in range(n_blocks):
            p = blk & 1
            if blk >= 2:
                pltpu.make_async_copy(out_hbm.at[:, pl.ds(0, _LANE)], prod[p],
                                      osem[p]).wait()
            lax.fori_loop(0, _LANE // nl, make_body(blk), 0)
            pltpu.async_copy(prod[p],
                             out_hbm.at[:, pl.ds(base + blk * _LANE, _LANE)],
                             osem[p])

        pltpu.make_async_copy(out_hbm.at[:, pl.ds(0, _LANE)], prod0,
                              osem0).wait()
        pltpu.make_async_copy(out_hbm.at[:, pl.ds(0, _LANE)], prod1,
                              osem1).wait()

    return gmf


_gmf = _build()


def kernel(user_indices, item_indices, user_table, item_table):
    uidx = user_indices.astype(jnp.int32)
    iidx = item_indices.astype(jnp.int32)
    outT = _gmf(uidx, iidx, user_table.T, item_table.T)
    return outT.T


# speedup vs baseline: 1.3644x; 1.0964x over previous
import functools

import jax
import jax.numpy as jnp
from jax import lax
from jax.experimental import pallas as pl
from jax.experimental.pallas import tpu as pltpu
from jax.experimental.pallas import tpu_sc as plsc

_EMBED = 64
_BATCH = 16384
_LANE = 128
_DEPTH = 6
_SB = 48


def _build():
    info = plsc.get_sparse_core_info()
    nc, ns, nl = info.num_cores, info.num_subcores, info.num_lanes
    nw = nc * ns
    b_per_w = _BATCH // nw
    n_sb = (b_per_w - 32) // _SB
    mesh = plsc.VectorSubcoreMesh(core_axis_name="c", subcore_axis_name="s")

    @functools.partial(
        pl.kernel,
        mesh=mesh,
        compiler_params=pltpu.CompilerParams(needs_layout_passes=False),
        out_type=jax.ShapeDtypeStruct((_EMBED, _BATCH), jnp.float32),
        scratch_types=[
            pltpu.VMEM((b_per_w + 32,), jnp.int32),
            pltpu.VMEM((b_per_w + 32,), jnp.int32),
            pltpu.VMEM((_DEPTH, _EMBED, _LANE), jnp.float32),
            pltpu.VMEM((_DEPTH, _EMBED, _LANE), jnp.float32),
            pltpu.VMEM((_EMBED, _LANE), jnp.float32),
            [pltpu.SemaphoreType.DMA] * _DEPTH,
        ],
    )
    def gmf(uidx_hbm, iidx_hbm, utab_hbm, itab_hbm, out_hbm,
            uidx_v, iidx_v, uw, iw, prod, wsem):
        wid = lax.axis_index("s") * nc + lax.axis_index("c")
        base = wid * b_per_w
        pltpu.sync_copy(uidx_hbm.at[pl.ds(base, b_per_w)],
                        uidx_v.at[pl.ds(0, b_per_w)])
        pltpu.sync_copy(iidx_hbm.at[pl.ds(base, b_per_w)],
                        iidx_v.at[pl.ds(0, b_per_w)])

        iotas = tuple(lax.iota(jnp.int32, nl) + j * nl
                      for j in range(_EMBED // nl))

        def issue(u_idx, i_idx, slot):
            uo = pl.multiple_of(lax.shift_right_logical(u_idx, 7) * _LANE, 128)
            io = pl.multiple_of(lax.shift_right_logical(i_idx, 7) * _LANE, 128)
            pltpu.async_copy(utab_hbm.at[:, pl.ds(uo, _LANE)], uw.at[slot],
                             wsem[slot])
            pltpu.async_copy(itab_hbm.at[:, pl.ds(io, _LANE)], iw.at[slot],
                             wsem[slot])

        def drain(slot):
            pltpu.make_async_copy(utab_hbm.at[:, pl.ds(0, _LANE)],
                                  uw.at[slot], wsem[slot]).wait()
            pltpu.make_async_copy(itab_hbm.at[:, pl.ds(0, _LANE)],
                                  iw.at[slot], wsem[slot]).wait()

        def extract(slot, u_val, i_val, ci):
            lu = jnp.full((nl,), lax.bitwise_and(u_val, 127), jnp.int32)
            li = jnp.full((nl,), lax.bitwise_and(i_val, 127), jnp.int32)
            lo = jnp.full((nl,), lax.bitwise_and(ci, 127), jnp.int32)
            for j in range(_EMBED // nl):
                gu = plsc.load_gather(uw.at[slot], [iotas[j], lu])
                gi = plsc.load_gather(iw.at[slot], [iotas[j], li])
                plsc.store_scatter(prod, [iotas[j], lo], gu * gi)

        def flush(ci):
            blk = lax.shift_right_logical(ci, 7)
            oo = pl.multiple_of(base + blk * _LANE, 128)
            pltpu.sync_copy(prod, out_hbm.at[:, pl.ds(oo, _LANE)])

        v0u = uidx_v[pl.ds(0, nl)]
        v0i = iidx_v[pl.ds(0, nl)]
        for s in range(_DEPTH):
            issue(v0u[s], v0i[s], s)

        def body(k, carry):
            go = k * _SB
            vu = tuple(uidx_v[pl.ds(go + 16 * g, nl)] for g in range(4))
            vi = tuple(iidx_v[pl.ds(go + 16 * g, nl)] for g in range(4))
            for e in range(_SB):
                ci = go + e
                g, lane = divmod(e, nl)
                s = e % _DEPTH
                drain(s)
                extract(s, vu[g][lane], vi[g][lane], ci)
                tg, tl = divmod(e + _DEPTH, nl)
                issue(vu[tg][tl], vi[tg][tl], s)

                @pl.when(lax.bitwise_and(ci, 127) == 127)
                def _():
                    flush(ci)
            return carry

        lax.fori_loop(0, n_sb, body, 0)

        t0 = n_sb * _SB
        vtu = tuple(uidx_v[pl.ds(t0 + 16 * g, nl)] for g in range(2))
        vti = tuple(iidx_v[pl.ds(t0 + 16 * g, nl)] for g in range(2))
        for e in range(b_per_w - t0):
            ci = t0 + e
            g, lane = divmod(e, nl)
            s = (t0 + e) % _DEPTH
            drain(s)
            extract(s, vtu[g][lane], vti[g][lane], jnp.int32(ci))
            if ci + _DEPTH < b_per_w:
                tg, tl = divmod(e + _DEPTH, nl)
                issue(vtu[tg][tl], vti[tg][tl], s)
        pltpu.sync_copy(prod,
                        out_hbm.at[:, pl.ds(base + (b_per_w - _LANE), _LANE)])

    return gmf


_gmf = _build()


def kernel(user_indices, item_indices, user_table, item_table):
    uidx = user_indices.astype(jnp.int32)
    iidx = item_indices.astype(jnp.int32)
    outT = _gmf(uidx, iidx, user_table.T, item_table.T)
    return outT.T
